# baseline (speedup 1.0000x reference)
import functools

import jax
import jax.numpy as jnp
from jax.experimental import pallas as pl
from jax.experimental.pallas import tpu as pltpu

_K = 2


def _se_attention(x, w1t, b1, w2t, b2, inv_hw):
    s = jnp.sum(x, axis=-1) * inv_hw + jnp.max(x, axis=-1)
    h = jnp.dot(s, w1t, preferred_element_type=jnp.float32)
    h = jnp.maximum(h + b1, 0.0)
    a = jnp.dot(h, w2t, preferred_element_type=jnp.float32)
    att = 1.0 + jax.nn.sigmoid(a + b2)
    return x * att[:, :, None]


def _se_kernel(xa_ref, xb_ref, w1t_ref, b1_ref, w2t_ref, b2_ref,
               oa_ref, ob_ref, *, inv_hw):
    w1t = w1t_ref[...]
    b1 = b1_ref[...]
    w2t = w2t_ref[...]
    b2 = b2_ref[...]
    oa_ref[...] = _se_attention(xa_ref[...], w1t, b1, w2t, b2, inv_hw)
    ob_ref[...] = _se_attention(xb_ref[...], w1t, b1, w2t, b2, inv_hw)


def kernel(x, w1, b1, w2, b2):
    B, C, H, W = x.shape
    Cr = w1.shape[0]
    HW = H * W
    inv_hw = 1.0 / HW
    half = B // 2
    n = half // _K

    x_k = x.reshape(B, C, HW)
    w1t = jnp.transpose(w1)
    b1_2d = b1.reshape(1, Cr)
    w2t = jnp.transpose(w2)
    b2_2d = b2.reshape(1, C)

    oa, ob = pl.pallas_call(
        functools.partial(_se_kernel, inv_hw=inv_hw),
        out_shape=(jax.ShapeDtypeStruct((half, C, HW), x.dtype),
                   jax.ShapeDtypeStruct((half, C, HW), x.dtype)),
        grid=(n,),
        in_specs=[
            pl.BlockSpec((_K, C, HW), lambda i: (i, 0, 0)),
            pl.BlockSpec((_K, C, HW), lambda i, _n=n: (i + _n, 0, 0)),
            pl.BlockSpec((C, Cr), lambda i: (0, 0)),
            pl.BlockSpec((1, Cr), lambda i: (0, 0)),
            pl.BlockSpec((Cr, C), lambda i: (0, 0)),
            pl.BlockSpec((1, C), lambda i: (0, 0)),
        ],
        out_specs=(pl.BlockSpec((_K, C, HW), lambda i: (i, 0, 0)),
                   pl.BlockSpec((_K, C, HW), lambda i: (i, 0, 0))),
        compiler_params=pltpu.CompilerParams(
            dimension_semantics=("arbitrary",),
            vmem_limit_bytes=60 << 20,
        ),
        cost_estimate=pl.CostEstimate(
            flops=int(4 * B * C * HW + 4 * B * C * Cr),
            transcendentals=int(B * C),
            bytes_accessed=int(2 * B * C * HW * 4),
        ),
    )(x_k, x_k, w1t, b1_2d, w2t, b2_2d)
    out_k = jnp.concatenate([oa, ob], axis=0)
    return out_k.reshape(B, C, H, W)


# speedup vs baseline: 1.2294x; 1.2294x over previous
import functools

import jax
import jax.numpy as jnp
from jax.experimental import pallas as pl
from jax.experimental.pallas import tpu as pltpu


def _se_kernel(x_ref, w1t_ref, b1_ref, w2t_ref, b2_ref, o_ref, *, inv_hw):
    x = x_ref[...].astype(jnp.float32)
    s = jnp.sum(x, axis=-1) * inv_hw + jnp.max(x, axis=-1)
    h = jnp.dot(s, w1t_ref[...], preferred_element_type=jnp.float32)
    h = jnp.maximum(h + b1_ref[...], 0.0)
    a = jnp.dot(h, w2t_ref[...], preferred_element_type=jnp.float32)
    att = 1.0 + jax.nn.sigmoid(a + b2_ref[...])
    o_ref[...] = (x * att[:, :, None]).astype(o_ref.dtype)


def kernel(x, w1, b1, w2, b2):
    B, C, H, W = x.shape
    Cr = w1.shape[0]
    HW = H * W
    inv_hw = 1.0 / HW

    elt = x.dtype.itemsize
    plane_bytes = C * HW * elt
    K = 1
    for cand in (2, 4):
        if B % cand == 0 and 4 * cand * plane_bytes <= 24 * 1024 * 1024:
            K = cand
            break

    x_k = x.reshape(B, C, HW)
    w1t = jnp.transpose(w1)
    b1_2d = b1.reshape(1, Cr)
    w2t = jnp.transpose(w2)
    b2_2d = b2.reshape(1, C)

    out_k = pl.pallas_call(
        functools.partial(_se_kernel, inv_hw=inv_hw),
        out_shape=jax.ShapeDtypeStruct((B, C, HW), x.dtype),
        grid=(B // K,),
        in_specs=[
            pl.BlockSpec((K, C, HW), lambda i: (i, 0, 0)),
            pl.BlockSpec((C, Cr), lambda i: (0, 0)),
            pl.BlockSpec((1, Cr), lambda i: (0, 0)),
            pl.BlockSpec((Cr, C), lambda i: (0, 0)),
            pl.BlockSpec((1, C), lambda i: (0, 0)),
        ],
        out_specs=pl.BlockSpec((K, C, HW), lambda i: (i, 0, 0)),
        compiler_params=pltpu.CompilerParams(
            dimension_semantics=("parallel",),
            vmem_limit_bytes=int(min(4 * K * plane_bytes + (4 << 20), 60 << 20)),
        ),
        cost_estimate=pl.CostEstimate(
            flops=int(4 * B * C * HW + 4 * B * C * Cr),
            transcendentals=int(B * C),
            bytes_accessed=int(2 * B * plane_bytes),
        ),
    )(x_k, w1t, b1_2d, w2t, b2_2d)
    return out_k.reshape(B, C, H, W)


# speedup vs baseline: 1.2425x; 1.0106x over previous
import functools

import jax
import jax.numpy as jnp
from jax.experimental import pallas as pl
from jax.experimental.pallas import tpu as pltpu


def _se_block(x_blk, w1t_blk, b1_blk, w2t_blk, b2_blk, o_blk, *, inv_hw):
    x = x_blk[...]
    s = jnp.sum(x, axis=-1) * inv_hw + jnp.max(x, axis=-1)
    h = jnp.dot(s, w1t_blk[...], preferred_element_type=jnp.float32)
    h = jnp.maximum(h + b1_blk[...], 0.0)
    a = jnp.dot(h, w2t_blk[...], preferred_element_type=jnp.float32)
    att = 1.0 + jax.nn.sigmoid(a + b2_blk[...])
    o_blk[...] = x * att[:, :, None]


def kernel(x, w1, b1, w2, b2):
    B, C, H, W = x.shape
    Cr = w1.shape[0]
    HW = H * W
    inv_hw = 1.0 / HW

    plane_bytes = C * HW * x.dtype.itemsize
    K = 1
    for cand in (4, 2):
        if B % cand == 0 and 4 * cand * plane_bytes <= 40 * 1024 * 1024:
            K = cand
            break

    x_k = x.reshape(B, C, HW)
    w1t = jnp.transpose(w1)
    b1_2d = b1.reshape(1, Cr)
    w2t = jnp.transpose(w2)
    b2_2d = b2.reshape(1, C)

    num_cores = getattr(jax.devices()[0], "num_cores", 1)
    mesh = pltpu.create_tensorcore_mesh("core", num_cores=num_cores)
    body = functools.partial(_se_block, inv_hw=inv_hw)

    def run(refs):
        x_ref, w1t_ref, b1_ref, w2t_ref, b2_ref, o_ref = refs

        @pl.core_map(
            mesh,
            compiler_params=pltpu.CompilerParams(
                vmem_limit_bytes=int(min(4 * K * plane_bytes + (4 << 20), 60 << 20)),
            ),
            cost_estimate=pl.CostEstimate(
                flops=int(4 * B * C * HW + 4 * B * C * Cr),
                transcendentals=int(B * C),
                bytes_accessed=int(2 * B * plane_bytes),
            ),
        )
        def _():
            pltpu.emit_pipeline(
                body,
                grid=(B // K,),
                in_specs=[
                    pl.BlockSpec((K, C, HW), lambda i: (i, 0, 0)),
                    pl.BlockSpec((C, Cr), lambda i: (0, 0)),
                    pl.BlockSpec((1, Cr), lambda i: (0, 0)),
                    pl.BlockSpec((Cr, C), lambda i: (0, 0)),
                    pl.BlockSpec((1, C), lambda i: (0, 0)),
                ],
                out_specs=[pl.BlockSpec((K, C, HW), lambda i: (i, 0, 0))],
                core_axis_name="core",
            )(x_ref, w1t_ref, b1_ref, w2t_ref, b2_ref, o_ref)

    init = (x_k, w1t, b1_2d, w2t, b2_2d, pl.empty((B, C, HW), x.dtype))
    *_, out_k = pl.run_state(run)(init)
    return out_k.reshape(B, C, H, W)


# speedup vs baseline: 1.2484x; 1.0047x over previous
import functools

import jax
import jax.numpy as jnp
from jax.experimental import pallas as pl
from jax.experimental.pallas import tpu as pltpu

_NBUF = 4
_K = 2


def _se_kernel(x_hbm, w1t_ref, b1_ref, w2t_ref, b2_ref, o_hbm,
               ibufs, obufs, isems, osems, *, inv_hw):
    i = pl.program_id(0)
    n = pl.num_programs(0)

    @pl.when(i == 0)
    def _():
        for j in range(min(_NBUF, n)):
            pltpu.make_async_copy(
                x_hbm.at[pl.ds(j * _K, _K)], ibufs.at[j], isems.at[j]
            ).start()

    slot = jax.lax.rem(i, _NBUF)
    pltpu.make_async_copy(
        x_hbm.at[pl.ds(i * _K, _K)], ibufs.at[slot], isems.at[slot]
    ).wait()

    @pl.when(i >= _NBUF)
    def _():
        prev = i - _NBUF
        pltpu.make_async_copy(
            obufs.at[slot], o_hbm.at[pl.ds(prev * _K, _K)], osems.at[slot]
        ).wait()

    x = ibufs[slot]
    s = jnp.sum(x, axis=-1) * inv_hw + jnp.max(x, axis=-1)
    h = jnp.dot(s, w1t_ref[...], preferred_element_type=jnp.float32)
    h = jnp.maximum(h + b1_ref[...], 0.0)
    a = jnp.dot(h, w2t_ref[...], preferred_element_type=jnp.float32)
    att = 1.0 + jax.nn.sigmoid(a + b2_ref[...])
    obufs[slot] = x * att[:, :, None]

    pltpu.make_async_copy(
        obufs.at[slot], o_hbm.at[pl.ds(i * _K, _K)], osems.at[slot]
    ).start()

    nxt = i + _NBUF

    @pl.when(nxt < n)
    def _():
        pltpu.make_async_copy(
            x_hbm.at[pl.ds(nxt * _K, _K)], ibufs.at[slot], isems.at[slot]
        ).start()

    @pl.when(i == n - 1)
    def _():
        for j in range(min(_NBUF, n)):
            step = n - min(_NBUF, n) + j
            pltpu.make_async_copy(
                obufs.at[step % _NBUF],
                o_hbm.at[pl.ds(step * _K, _K)],
                osems.at[step % _NBUF],
            ).wait()


def kernel(x, w1, b1, w2, b2):
    B, C, H, W = x.shape
    Cr = w1.shape[0]
    HW = H * W
    inv_hw = 1.0 / HW

    x_k = x.reshape(B, C, HW)
    w1t = jnp.transpose(w1)
    b1_2d = b1.reshape(1, Cr)
    w2t = jnp.transpose(w2)
    b2_2d = b2.reshape(1, C)

    out_k = pl.pallas_call(
        functools.partial(_se_kernel, inv_hw=inv_hw),
        out_shape=jax.ShapeDtypeStruct((B, C, HW), x.dtype),
        grid=(B // _K,),
        in_specs=[
            pl.BlockSpec(memory_space=pl.ANY),
            pl.BlockSpec((C, Cr), lambda i: (0, 0)),
            pl.BlockSpec((1, Cr), lambda i: (0, 0)),
            pl.BlockSpec((Cr, C), lambda i: (0, 0)),
            pl.BlockSpec((1, C), lambda i: (0, 0)),
        ],
        out_specs=pl.BlockSpec(memory_space=pl.ANY),
        scratch_shapes=[
            pltpu.VMEM((_NBUF, _K, C, HW), jnp.float32),
            pltpu.VMEM((_NBUF, _K, C, HW), jnp.float32),
            pltpu.SemaphoreType.DMA((_NBUF,)),
            pltpu.SemaphoreType.DMA((_NBUF,)),
        ],
        compiler_params=pltpu.CompilerParams(
            dimension_semantics=("arbitrary",),
            vmem_limit_bytes=48 << 20,
        ),
        cost_estimate=pl.CostEstimate(
            flops=int(4 * B * C * HW + 4 * B * C * Cr),
            transcendentals=int(B * C),
            bytes_accessed=int(2 * B * C * HW * 4),
        ),
    )(x_k, w1t, b1_2d, w2t, b2_2d)
    return out_k.reshape(B, C, H, W)
